# Initial kernel scaffold; baseline (speedup 1.0000x reference)
import jax
import jax.numpy as jnp
from jax.experimental import pallas as pl


def kernel(xyz, features, W1, b1, W2, b2, W3, b3):
    raise NotImplementedError("write your pallas kernel here")


# speedup vs baseline: 10.1337x; 10.1337x over previous
import functools

import jax
import jax.numpy as jnp
from jax import lax
from jax.experimental import pallas as pl
from jax.experimental.pallas import tpu as pltpu
from jax.experimental.pallas import tpu_sc as plsc

_B = 8
_N = 4096
_S = 512
_K = 32
_CF = 64
_CP = 80
_TQ = 256

_NW = 32
_ROWS = _B * _S * _K
_RPW = _ROWS // _NW
_CH = 128
_NCH = _RPW // _CH
_GRP = 4
_NGRP = _NCH // _GRP


def _fps_body(xyzt_ref, newt_ref):
    X = xyzt_ref[0]
    Y = xyzt_ref[1]
    Z = xyzt_ref[2]
    lane = lax.broadcasted_iota(jnp.int32, (_B, _N), 1)
    scol = lax.broadcasted_iota(jnp.int32, (_B, _S), 1)

    def body(i, carry):
        dist, f, nx, ny, nz = carry
        oh = lane == f
        cx = jnp.sum(jnp.where(oh, X, 0.0), axis=1, keepdims=True)
        cy = jnp.sum(jnp.where(oh, Y, 0.0), axis=1, keepdims=True)
        cz = jnp.sum(jnp.where(oh, Z, 0.0), axis=1, keepdims=True)
        sm = scol == i
        nx = nx + jnp.where(sm, cx, 0.0)
        ny = ny + jnp.where(sm, cy, 0.0)
        nz = nz + jnp.where(sm, cz, 0.0)
        d = (X - cx) ** 2 + (Y - cy) ** 2 + (Z - cz) ** 2
        dist = jnp.minimum(dist, d)
        m = jnp.max(dist, axis=1, keepdims=True)
        cand = jnp.where(dist == m, lane, _N)
        f = jnp.min(cand, axis=1, keepdims=True)
        return dist, f, nx, ny, nz

    init = (
        jnp.full((_B, _N), 1e10, jnp.float32),
        jnp.zeros((_B, 1), jnp.int32),
        jnp.zeros((_B, _S), jnp.float32),
        jnp.zeros((_B, _S), jnp.float32),
        jnp.zeros((_B, _S), jnp.float32),
    )
    _, _, nx, ny, nz = lax.fori_loop(0, _S, body, init)
    newt_ref[0] = nx
    newt_ref[1] = ny
    newt_ref[2] = nz


def _fps(xyzt):
    return pl.pallas_call(
        _fps_body,
        out_shape=jax.ShapeDtypeStruct((3, _B, _S), jnp.float32),
    )(xyzt)


def _knn_body(xyzt_ref, q_ref, idx_ref):
    b = pl.program_id(0)
    X = xyzt_ref[0, 0]
    Y = xyzt_ref[1, 0]
    Z = xyzt_ref[2, 0]
    qx = q_ref[0, 0]
    qy = q_ref[1, 0]
    qz = q_ref[2, 0]
    d = (X - qx) ** 2 + (Y - qy) ** 2 + (Z - qz) ** 2
    lane = lax.broadcasted_iota(jnp.int32, (_TQ, _N), 1)
    kcol = lax.broadcasted_iota(jnp.int32, (_TQ, _K), 1)
    base = b * _N

    def body(it, carry):
        d, acc = carry
        m = jnp.min(d, axis=1, keepdims=True)
        cand = jnp.where(d == m, lane, _N)
        j = jnp.min(cand, axis=1, keepdims=True)
        acc = acc + jnp.where(kcol == it, j + base, 0)
        d = jnp.where(lane == j, 1e30, d)
        return d, acc

    _, acc = lax.fori_loop(0, _K, body, (d, jnp.zeros((_TQ, _K), jnp.int32)))
    idx_ref[...] = acc[None]


def _knn(xyzt, q4):
    return pl.pallas_call(
        _knn_body,
        grid=(_B, _S // _TQ),
        in_specs=[
            pl.BlockSpec((3, 1, 1, _N), lambda b, t: (0, b, 0, 0)),
            pl.BlockSpec((3, 1, _TQ, 1), lambda b, t: (0, b, t, 0)),
        ],
        out_specs=pl.BlockSpec((1, _TQ, _K), lambda b, t: (b, t, 0)),
        out_shape=jax.ShapeDtypeStruct((_B, _S, _K), jnp.int32),
    )(xyzt[:, :, None, :], q4)


def _sc_gather_body(table_hbm, idx_hbm, out_hbm, idx_v, rows_v, sem):
    cid = lax.axis_index("c")
    sid = lax.axis_index("s")
    wid = sid * 2 + cid
    base = wid * _RPW
    pltpu.sync_copy(idx_hbm.at[pl.ds(wid * _NCH, _NCH)], idx_v)

    def group(g, _):
        copies = []
        for q in range(_GRP):
            copies.append(
                pltpu.async_copy(
                    table_hbm.at[idx_v.at[g * _GRP + q]],
                    rows_v.at[pl.ds(q * _CH, _CH)],
                    sem,
                )
            )
        for c in copies:
            c.wait()
        pltpu.sync_copy(
            rows_v, out_hbm.at[pl.ds(base + g * (_GRP * _CH), _GRP * _CH)]
        )
        return 0

    lax.fori_loop(0, _NGRP, group, 0)


def _sc_gather(table, idx2d):
    mesh = plsc.VectorSubcoreMesh(core_axis_name="c", subcore_axis_name="s")
    f = functools.partial(
        pl.kernel,
        out_type=jax.ShapeDtypeStruct((_ROWS, _CP), jnp.float32),
        mesh=mesh,
        scratch_types=[
            pltpu.VMEM((_NCH, _CH), jnp.int32),
            pltpu.VMEM((_GRP * _CH, _CP), jnp.float32),
            pltpu.SemaphoreType.DMA,
        ],
        compiler_params=pltpu.CompilerParams(use_tc_tiling_on_sc=False),
    )(_sc_gather_body)
    return f(table, idx2d)


def _mlp_body(rows_ref, nxp_ref, w1_ref, b1_ref, w2_ref, b2_ref, w3_ref,
              b3_ref, out_ref):
    g = rows_ref.shape[0]
    x = rows_ref[...] - nxp_ref[...]
    x = x.reshape(g * _K, _CP)
    h = jnp.dot(x, w1_ref[...], preferred_element_type=jnp.float32)
    h = jnp.maximum(h + b1_ref[...], 0.0)
    h = jnp.dot(h, w2_ref[...], preferred_element_type=jnp.float32)
    h = jnp.maximum(h + b2_ref[...], 0.0)
    h = jnp.dot(h, w3_ref[...], preferred_element_type=jnp.float32)
    h = jnp.maximum(h + b3_ref[...], 0.0)
    out_ref[...] = jnp.max(h.reshape(g, _K, 128), axis=1)


def _mlp(rows3, nxp, w1p, b1, w2, b2, w3, b3):
    G = 64
    grid = (_B * _S) // G
    return pl.pallas_call(
        _mlp_body,
        grid=(grid,),
        in_specs=[
            pl.BlockSpec((G, _K, _CP), lambda i: (i, 0, 0)),
            pl.BlockSpec((G, 1, _CP), lambda i: (i, 0, 0)),
            pl.BlockSpec((_CP, 64), lambda i: (0, 0)),
            pl.BlockSpec((1, 64), lambda i: (0, 0)),
            pl.BlockSpec((64, 64), lambda i: (0, 0)),
            pl.BlockSpec((1, 64), lambda i: (0, 0)),
            pl.BlockSpec((64, 128), lambda i: (0, 0)),
            pl.BlockSpec((1, 128), lambda i: (0, 0)),
        ],
        out_specs=pl.BlockSpec((G, 128), lambda i: (i, 0)),
        out_shape=jax.ShapeDtypeStruct((_B * _S, 128), jnp.float32),
    )(rows3, nxp, w1p, b1, w2, b2, w3, b3)


def kernel(xyz, features, W1, b1, W2, b2, W3, b3):
    f32 = jnp.float32
    xyzt = jnp.transpose(xyz, (2, 0, 1))
    newt = _fps(xyzt)
    new_xyz = jnp.transpose(newt, (1, 2, 0))
    idx = _knn(xyzt, newt[..., None])
    idx2d = idx.reshape(_ROWS // _CH, _CH)

    table = jnp.concatenate(
        [xyz, features, jnp.zeros((_B, _N, _CP - 3 - _CF), f32)], axis=-1
    ).reshape(_B * _N, _CP)
    rows = _sc_gather(table, idx2d)
    rows3 = rows.reshape(_B * _S, _K, _CP)

    nxp = jnp.concatenate(
        [new_xyz, jnp.zeros((_B, _S, _CP - 3), f32)], axis=-1
    ).reshape(_B * _S, 1, _CP)
    w1p = jnp.concatenate([W1, jnp.zeros((_CP - 67, 64), f32)], axis=0)

    feats = _mlp(rows3, nxp, w1p, b1[None], W2, b2[None], W3, b3[None])
    return new_xyz, feats.reshape(_B, _S, 128)


# speedup vs baseline: 15.4053x; 1.5202x over previous
import functools

import jax
import jax.numpy as jnp
from jax import lax
from jax.experimental import pallas as pl
from jax.experimental.pallas import tpu as pltpu
from jax.experimental.pallas import tpu_sc as plsc

_B = 8
_N = 4096
_S = 512
_K = 32
_CF = 64
_CP = 80
_TQ = 256

_NW = 32
_ROWS = _B * _S * _K
_RPW = _ROWS // _NW
_CH = 128
_NCH = _RPW // _CH
_GRP = 4
_NGRP = _NCH // _GRP


def _fps_body(xyzt_ref, newt_ref):
    X = xyzt_ref[0]
    Y = xyzt_ref[1]
    Z = xyzt_ref[2]
    lane = lax.broadcasted_iota(jnp.int32, (_B, _N), 1)
    scol = lax.broadcasted_iota(jnp.int32, (_B, _S), 1)

    def body(i, carry):
        dist, f, nx, ny, nz = carry
        oh = lane == f
        cx = jnp.sum(jnp.where(oh, X, 0.0), axis=1, keepdims=True)
        cy = jnp.sum(jnp.where(oh, Y, 0.0), axis=1, keepdims=True)
        cz = jnp.sum(jnp.where(oh, Z, 0.0), axis=1, keepdims=True)
        sm = scol == i
        nx = nx + jnp.where(sm, cx, 0.0)
        ny = ny + jnp.where(sm, cy, 0.0)
        nz = nz + jnp.where(sm, cz, 0.0)
        d = (X - cx) ** 2 + (Y - cy) ** 2 + (Z - cz) ** 2
        dist = jnp.minimum(dist, d)
        m = jnp.max(dist, axis=1, keepdims=True)
        cand = jnp.where(dist == m, lane, _N)
        f = jnp.min(cand, axis=1, keepdims=True)
        return dist, f, nx, ny, nz

    init = (
        jnp.full((_B, _N), 1e10, jnp.float32),
        jnp.zeros((_B, 1), jnp.int32),
        jnp.zeros((_B, _S), jnp.float32),
        jnp.zeros((_B, _S), jnp.float32),
        jnp.zeros((_B, _S), jnp.float32),
    )
    _, _, nx, ny, nz = lax.fori_loop(0, _S, body, init)
    newt_ref[0] = nx
    newt_ref[1] = ny
    newt_ref[2] = nz


def _fps(xyzt):
    return pl.pallas_call(
        _fps_body,
        out_shape=jax.ShapeDtypeStruct((3, _B, _S), jnp.float32),
    )(xyzt)


_NCHK = _N // 128
_DEPTH = 4


def _knn_body(xyzt_ref, q_ref, idx_ref):
    b = pl.program_id(0)
    X = xyzt_ref[0, 0]
    Y = xyzt_ref[1, 0]
    Z = xyzt_ref[2, 0]
    qx = q_ref[0, 0]
    qy = q_ref[1, 0]
    qz = q_ref[2, 0]
    d = (X - qx) ** 2 + (Y - qy) ** 2 + (Z - qz) ** 2

    i32 = jnp.int32
    f32 = jnp.float32
    BIGV = f32(3.0e38)
    BIGI = i32(2**30)
    lane = lax.broadcasted_iota(i32, (_TQ, 128), 1)
    kcol = lax.broadcasted_iota(i32, (_TQ, _K), 1)
    base = b * _N

    V = [jnp.full((_TQ, 128), BIGV, f32) for _ in range(_DEPTH)]
    C = [jnp.full((_TQ, 128), _NCHK, i32) for _ in range(_DEPTH)]
    for c in range(_NCHK):
        cv = d[:, c * 128:(c + 1) * 128]
        cc = jnp.full((_TQ, 128), c, i32)
        for s in range(_DEPTH):
            p = cv < V[s]
            V[s], cv = jnp.where(p, cv, V[s]), jnp.where(p, V[s], cv)
            C[s], cc = jnp.where(p, cc, C[s]), jnp.where(p, C[s], cc)

    def refill_next(LV, LC):
        rv = jnp.full((_TQ, 128), BIGV, f32)
        rc = jnp.full((_TQ, 128), _NCHK, i32)
        for c in range(_NCHK):
            x = d[:, c * 128:(c + 1) * 128]
            elig = (x > LV) | ((x == LV) & (c > LC))
            xv = jnp.where(elig, x, BIGV)
            better = (xv < rv)
            rv = jnp.where(better, xv, rv)
            rc = jnp.where(better, jnp.full((_TQ, 128), c, i32), rc)
        return rv, rc

    def body(r, carry):
        acc, H, HC, D, BD, LV, LC = carry
        m = jnp.min(H, axis=1, keepdims=True)
        gcand = jnp.where(H == m, HC * 128 + lane, BIGI)
        j = jnp.min(gcand, axis=1, keepdims=True)
        acc = acc + jnp.where(kcol == r, j + base, 0)
        win = gcand == j
        Dn = D + win.astype(i32)
        LV = jnp.where(win, H, LV)
        LC = jnp.where(win, HC, LC)
        nv = jnp.full((_TQ, 128), BIGV, f32)
        nc = jnp.full((_TQ, 128), _NCHK, i32)
        for s in range(_DEPTH - 1, 0, -1):
            nv = jnp.where(Dn == s, V[s], nv)
            nc = jnp.where(Dn == s, C[s], nc)
        H = jnp.where(win, nv, H)
        HC = jnp.where(win, nc, HC)
        need = win & (Dn >= BD)

        def do_refill(ops):
            H, HC, BD = ops
            rv, rc = refill_next(LV, LC)
            H = jnp.where(need, rv, H)
            HC = jnp.where(need, rc, HC)
            BD = BD + need.astype(i32)
            return H, HC, BD

        H, HC, BD = lax.cond(jnp.sum(need.astype(i32)) > 0,
                             do_refill, lambda ops: ops, (H, HC, BD))
        return acc, H, HC, Dn, BD, LV, LC

    init = (
        jnp.zeros((_TQ, _K), i32),
        V[0], C[0],
        jnp.zeros((_TQ, 128), i32),
        jnp.full((_TQ, 128), _DEPTH, i32),
        jnp.full((_TQ, 128), -BIGV, f32),
        jnp.zeros((_TQ, 128), i32),
    )
    acc = lax.fori_loop(0, _K, body, init)[0]
    idx_ref[...] = acc[None]


def _knn(xyzt, q4):
    return pl.pallas_call(
        _knn_body,
        grid=(_B, _S // _TQ),
        in_specs=[
            pl.BlockSpec((3, 1, 1, _N), lambda b, t: (0, b, 0, 0)),
            pl.BlockSpec((3, 1, _TQ, 1), lambda b, t: (0, b, t, 0)),
        ],
        out_specs=pl.BlockSpec((1, _TQ, _K), lambda b, t: (b, t, 0)),
        out_shape=jax.ShapeDtypeStruct((_B, _S, _K), jnp.int32),
    )(xyzt[:, :, None, :], q4)


def _sc_gather_body(table_hbm, idx_hbm, out_hbm, idx_v, rows_v, sem):
    cid = lax.axis_index("c")
    sid = lax.axis_index("s")
    wid = sid * 2 + cid
    base = wid * _RPW
    pltpu.sync_copy(idx_hbm.at[pl.ds(wid * _NCH, _NCH)], idx_v)

    def group(g, _):
        copies = []
        for q in range(_GRP):
            copies.append(
                pltpu.async_copy(
                    table_hbm.at[idx_v.at[g * _GRP + q]],
                    rows_v.at[pl.ds(q * _CH, _CH)],
                    sem,
                )
            )
        for c in copies:
            c.wait()
        pltpu.sync_copy(
            rows_v, out_hbm.at[pl.ds(base + g * (_GRP * _CH), _GRP * _CH)]
        )
        return 0

    lax.fori_loop(0, _NGRP, group, 0)


def _sc_gather(table, idx2d):
    mesh = plsc.VectorSubcoreMesh(core_axis_name="c", subcore_axis_name="s")
    f = functools.partial(
        pl.kernel,
        out_type=jax.ShapeDtypeStruct((_ROWS, _CP), jnp.float32),
        mesh=mesh,
        scratch_types=[
            pltpu.VMEM((_NCH, _CH), jnp.int32),
            pltpu.VMEM((_GRP * _CH, _CP), jnp.float32),
            pltpu.SemaphoreType.DMA,
        ],
        compiler_params=pltpu.CompilerParams(use_tc_tiling_on_sc=False),
    )(_sc_gather_body)
    return f(table, idx2d)


def _mlp_body(rows_ref, nxp_ref, w1_ref, b1_ref, w2_ref, b2_ref, w3_ref,
              b3_ref, out_ref):
    g = rows_ref.shape[0]
    x = rows_ref[...] - nxp_ref[...]
    x = x.reshape(g * _K, _CP)
    h = jnp.dot(x, w1_ref[...], preferred_element_type=jnp.float32)
    h = jnp.maximum(h + b1_ref[...], 0.0)
    h = jnp.dot(h, w2_ref[...], preferred_element_type=jnp.float32)
    h = jnp.maximum(h + b2_ref[...], 0.0)
    h = jnp.dot(h, w3_ref[...], preferred_element_type=jnp.float32)
    h = jnp.maximum(h + b3_ref[...], 0.0)
    out_ref[...] = jnp.max(h.reshape(g, _K, 128), axis=1)


def _mlp(rows3, nxp, w1p, b1, w2, b2, w3, b3):
    G = 64
    grid = (_B * _S) // G
    return pl.pallas_call(
        _mlp_body,
        grid=(grid,),
        in_specs=[
            pl.BlockSpec((G, _K, _CP), lambda i: (i, 0, 0)),
            pl.BlockSpec((G, 1, _CP), lambda i: (i, 0, 0)),
            pl.BlockSpec((_CP, 64), lambda i: (0, 0)),
            pl.BlockSpec((1, 64), lambda i: (0, 0)),
            pl.BlockSpec((64, 64), lambda i: (0, 0)),
            pl.BlockSpec((1, 64), lambda i: (0, 0)),
            pl.BlockSpec((64, 128), lambda i: (0, 0)),
            pl.BlockSpec((1, 128), lambda i: (0, 0)),
        ],
        out_specs=pl.BlockSpec((G, 128), lambda i: (i, 0)),
        out_shape=jax.ShapeDtypeStruct((_B * _S, 128), jnp.float32),
    )(rows3, nxp, w1p, b1, w2, b2, w3, b3)


def kernel(xyz, features, W1, b1, W2, b2, W3, b3):
    f32 = jnp.float32
    xyzt = jnp.transpose(xyz, (2, 0, 1))
    newt = _fps(xyzt)
    new_xyz = jnp.transpose(newt, (1, 2, 0))
    idx = _knn(xyzt, newt[..., None])
    idx2d = idx.reshape(_ROWS // _CH, _CH)

    table = jnp.concatenate(
        [xyz, features, jnp.zeros((_B, _N, _CP - 3 - _CF), f32)], axis=-1
    ).reshape(_B * _N, _CP)
    rows = _sc_gather(table, idx2d)
    rows3 = rows.reshape(_B * _S, _K, _CP)

    nxp = jnp.concatenate(
        [new_xyz, jnp.zeros((_B, _S, _CP - 3), f32)], axis=-1
    ).reshape(_B * _S, 1, _CP)
    w1p = jnp.concatenate([W1, jnp.zeros((_CP - 67, 64), f32)], axis=0)

    feats = _mlp(rows3, nxp, w1p, b1[None], W2, b2[None], W3, b3[None])
    return new_xyz, feats.reshape(_B, _S, 128)


# speedup vs baseline: 18.0575x; 1.1722x over previous
import functools

import jax
import jax.numpy as jnp
from jax import lax
from jax.experimental import pallas as pl
from jax.experimental.pallas import tpu as pltpu
from jax.experimental.pallas import tpu_sc as plsc

_B = 8
_N = 4096
_S = 512
_K = 32
_CF = 64
_CP = 80
_TQ = 256

_NW = 32
_ROWS = _B * _S * _K
_RPW = _ROWS // _NW
_CH = 128
_NCH = _RPW // _CH
_GRP = 4
_NGRP = _NCH // _GRP


def _fps_body(xyzt_ref, newt_ref):
    X = xyzt_ref[0]
    Y = xyzt_ref[1]
    Z = xyzt_ref[2]
    lane = lax.broadcasted_iota(jnp.int32, (_B, _N), 1)
    scol = lax.broadcasted_iota(jnp.int32, (_B, _S), 1)

    def body(i, carry):
        dist, f, nx, ny, nz = carry
        oh = lane == f
        cx = jnp.sum(jnp.where(oh, X, 0.0), axis=1, keepdims=True)
        cy = jnp.sum(jnp.where(oh, Y, 0.0), axis=1, keepdims=True)
        cz = jnp.sum(jnp.where(oh, Z, 0.0), axis=1, keepdims=True)
        sm = scol == i
        nx = nx + jnp.where(sm, cx, 0.0)
        ny = ny + jnp.where(sm, cy, 0.0)
        nz = nz + jnp.where(sm, cz, 0.0)
        d = (X - cx) ** 2 + (Y - cy) ** 2 + (Z - cz) ** 2
        dist = jnp.minimum(dist, d)
        m = jnp.max(dist, axis=1, keepdims=True)
        cand = jnp.where(dist == m, lane, _N)
        f = jnp.min(cand, axis=1, keepdims=True)
        return dist, f, nx, ny, nz

    init = (
        jnp.full((_B, _N), 1e10, jnp.float32),
        jnp.zeros((_B, 1), jnp.int32),
        jnp.zeros((_B, _S), jnp.float32),
        jnp.zeros((_B, _S), jnp.float32),
        jnp.zeros((_B, _S), jnp.float32),
    )
    _, _, nx, ny, nz = lax.fori_loop(0, _S, body, init)
    newt_ref[0] = nx
    newt_ref[1] = ny
    newt_ref[2] = nz


def _fps(xyzt):
    return pl.pallas_call(
        _fps_body,
        out_shape=jax.ShapeDtypeStruct((3, _B, _S), jnp.float32),
    )(xyzt)


_NCHK = _N // 128
_DEPTH = 4


def _knn_body(xyzt_ref, q_ref, idx_ref):
    b = pl.program_id(0)
    Xc = xyzt_ref[0, 0]
    Yc = xyzt_ref[1, 0]
    Zc = xyzt_ref[2, 0]
    qx = q_ref[0, 0]
    qy = q_ref[1, 0]
    qz = q_ref[2, 0]
    d = (Xc - qx) ** 2 + (Yc - qy) ** 2 + (Zc - qz) ** 2

    i32 = jnp.int32
    f32 = jnp.float32
    BIGV = f32(3.0e38)
    BIGI = i32(2**30)
    pos = lax.broadcasted_iota(i32, (128, _TQ), 0)
    krow = lax.broadcasted_iota(i32, (_K, _TQ), 0)
    base = b * _N

    V = [jnp.full((128, _TQ), BIGV, f32) for _ in range(_DEPTH)]
    C = [jnp.full((128, _TQ), _NCHK, i32) for _ in range(_DEPTH)]
    for c in range(_NCHK):
        cv = d[c * 128:(c + 1) * 128, :]
        cc = jnp.full((128, _TQ), c, i32)
        for s in range(_DEPTH):
            p = cv < V[s]
            V[s], cv = jnp.where(p, cv, V[s]), jnp.where(p, V[s], cv)
            C[s], cc = jnp.where(p, cc, C[s]), jnp.where(p, C[s], cc)

    def refill_next(LV, LC):
        rv = jnp.full((128, _TQ), BIGV, f32)
        rc = jnp.full((128, _TQ), _NCHK, i32)
        for c in range(_NCHK):
            x = d[c * 128:(c + 1) * 128, :]
            elig = (x > LV) | ((x == LV) & (c > LC))
            xv = jnp.where(elig, x, BIGV)
            better = (xv < rv)
            rv = jnp.where(better, xv, rv)
            rc = jnp.where(better, jnp.full((128, _TQ), c, i32), rc)
        return rv, rc

    def body(r, carry):
        acc, H, HC, D, BD, LV, LC = carry
        m = jnp.min(H, axis=0, keepdims=True)
        gcand = jnp.where(H == m, HC * 128 + pos, BIGI)
        j = jnp.min(gcand, axis=0, keepdims=True)
        acc = acc + jnp.where(krow == r, j + base, 0)
        win = gcand == j
        Dn = D + win.astype(i32)
        LV = jnp.where(win, H, LV)
        LC = jnp.where(win, HC, LC)
        nv = jnp.full((128, _TQ), BIGV, f32)
        nc = jnp.full((128, _TQ), _NCHK, i32)
        for s in range(_DEPTH - 1, 0, -1):
            nv = jnp.where(Dn == s, V[s], nv)
            nc = jnp.where(Dn == s, C[s], nc)
        H = jnp.where(win, nv, H)
        HC = jnp.where(win, nc, HC)
        need = win & (Dn >= BD)

        def do_refill(ops):
            H, HC, BD = ops
            rv, rc = refill_next(LV, LC)
            H = jnp.where(need, rv, H)
            HC = jnp.where(need, rc, HC)
            BD = BD + need.astype(i32)
            return H, HC, BD

        H, HC, BD = lax.cond(jnp.sum(need.astype(i32)) > 0,
                             do_refill, lambda ops: ops, (H, HC, BD))
        return acc, H, HC, Dn, BD, LV, LC

    init = (
        jnp.zeros((_K, _TQ), i32),
        V[0], C[0],
        jnp.zeros((128, _TQ), i32),
        jnp.full((128, _TQ), _DEPTH, i32),
        jnp.full((128, _TQ), -BIGV, f32),
        jnp.zeros((128, _TQ), i32),
    )
    acc = lax.fori_loop(0, _K, body, init)[0]
    idx_ref[...] = acc[None]


def _knn(xyzt, q3):
    return pl.pallas_call(
        _knn_body,
        grid=(_B, _S // _TQ),
        in_specs=[
            pl.BlockSpec((3, 1, _N, 1), lambda b, t: (0, b, 0, 0)),
            pl.BlockSpec((3, 1, 1, _TQ), lambda b, t: (0, b, 0, t)),
        ],
        out_specs=pl.BlockSpec((1, _K, _TQ), lambda b, t: (b, 0, t)),
        out_shape=jax.ShapeDtypeStruct((_B, _K, _S), jnp.int32),
    )(xyzt[..., None], q3[:, :, None, :])


def _sc_gather_body(table_hbm, idx_hbm, out_hbm, idx_v, rows_v, sem):
    cid = lax.axis_index("c")
    sid = lax.axis_index("s")
    wid = sid * 2 + cid
    base = wid * _RPW
    pltpu.sync_copy(idx_hbm.at[pl.ds(wid * _NCH, _NCH)], idx_v)

    def group(g, _):
        copies = []
        for q in range(_GRP):
            copies.append(
                pltpu.async_copy(
                    table_hbm.at[idx_v.at[g * _GRP + q]],
                    rows_v.at[pl.ds(q * _CH, _CH)],
                    sem,
                )
            )
        for c in copies:
            c.wait()
        pltpu.sync_copy(
            rows_v, out_hbm.at[pl.ds(base + g * (_GRP * _CH), _GRP * _CH)]
        )
        return 0

    lax.fori_loop(0, _NGRP, group, 0)


def _sc_gather(table, idx2d):
    mesh = plsc.VectorSubcoreMesh(core_axis_name="c", subcore_axis_name="s")
    f = functools.partial(
        pl.kernel,
        out_type=jax.ShapeDtypeStruct((_ROWS, _CP), jnp.float32),
        mesh=mesh,
        scratch_types=[
            pltpu.VMEM((_NCH, _CH), jnp.int32),
            pltpu.VMEM((_GRP * _CH, _CP), jnp.float32),
            pltpu.SemaphoreType.DMA,
        ],
        compiler_params=pltpu.CompilerParams(use_tc_tiling_on_sc=False),
    )(_sc_gather_body)
    return f(table, idx2d)


def _mlp_body(rows_ref, nxp_ref, w1_ref, b1_ref, w2_ref, b2_ref, w3_ref,
              b3_ref, out_ref):
    g = rows_ref.shape[0]
    x = rows_ref[...] - nxp_ref[...]
    x = x.reshape(g * _K, _CP)
    h = jnp.dot(x, w1_ref[...], preferred_element_type=jnp.float32)
    h = jnp.maximum(h + b1_ref[...], 0.0)
    h = jnp.dot(h, w2_ref[...], preferred_element_type=jnp.float32)
    h = jnp.maximum(h + b2_ref[...], 0.0)
    h = jnp.dot(h, w3_ref[...], preferred_element_type=jnp.float32)
    h = jnp.maximum(h + b3_ref[...], 0.0)
    out_ref[...] = jnp.max(h.reshape(g, _K, 128), axis=1)


def _mlp(rows3, nxp, w1p, b1, w2, b2, w3, b3):
    G = 64
    grid = (_B * _S) // G
    return pl.pallas_call(
        _mlp_body,
        grid=(grid,),
        in_specs=[
            pl.BlockSpec((G, _K, _CP), lambda i: (i, 0, 0)),
            pl.BlockSpec((G, 1, _CP), lambda i: (i, 0, 0)),
            pl.BlockSpec((_CP, 64), lambda i: (0, 0)),
            pl.BlockSpec((1, 64), lambda i: (0, 0)),
            pl.BlockSpec((64, 64), lambda i: (0, 0)),
            pl.BlockSpec((1, 64), lambda i: (0, 0)),
            pl.BlockSpec((64, 128), lambda i: (0, 0)),
            pl.BlockSpec((1, 128), lambda i: (0, 0)),
        ],
        out_specs=pl.BlockSpec((G, 128), lambda i: (i, 0)),
        out_shape=jax.ShapeDtypeStruct((_B * _S, 128), jnp.float32),
    )(rows3, nxp, w1p, b1, w2, b2, w3, b3)


def kernel(xyz, features, W1, b1, W2, b2, W3, b3):
    f32 = jnp.float32
    xyzt = jnp.transpose(xyz, (2, 0, 1))
    newt = _fps(xyzt)
    new_xyz = jnp.transpose(newt, (1, 2, 0))
    idxk = _knn(xyzt, newt)
    idx2d = jnp.transpose(idxk, (0, 2, 1)).reshape(_ROWS // _CH, _CH)

    table = jnp.concatenate(
        [xyz, features, jnp.zeros((_B, _N, _CP - 3 - _CF), f32)], axis=-1
    ).reshape(_B * _N, _CP)
    rows = _sc_gather(table, idx2d)
    rows3 = rows.reshape(_B * _S, _K, _CP)

    nxp = jnp.concatenate(
        [new_xyz, jnp.zeros((_B, _S, _CP - 3), f32)], axis=-1
    ).reshape(_B * _S, 1, _CP)
    w1p = jnp.concatenate([W1, jnp.zeros((_CP - 67, 64), f32)], axis=0)

    feats = _mlp(rows3, nxp, w1p, b1[None], W2, b2[None], W3, b3[None])
    return new_xyz, feats.reshape(_B, _S, 128)
